# baseline (speedup 1.0000x reference)
import functools
import jax
import jax.numpy as jnp
from jax import lax
from jax.experimental import pallas as pl
from jax.experimental.pallas import tpu as pltpu
from jax.experimental.pallas import tpu_sc as plsc

N_POS = 1024
SIZE = 192
SUB = SIZE // 6
B, L = 4096, 200
N = B * L

NC, NS, LANES = 2, 16, 16
NW = NC * NS
PER_W = N // NW
C = 256
CHUNKS = PER_W // C

_SCALES = (float(N_POS),) * 5 + (float(5 * N_POS),)
_OFFS = (0, N_POS, 0, N_POS, 2 * N_POS, 3 * N_POS)
_MAXF = float(N_POS - 1)


def _body(xmin, ymin, xmax, ymax, width, height, fp, lp,
          tcat, fpe, lpe, out,
          cb0, cb1, pf0, pf1, pq0, pq1, ib0, ib1, gb0, gb1,
          fpev, lpev, shared,
          sin0, sin1, sg0, sg1, sout0, sout1):
    coords = (xmin, ymin, xmax, ymax, width, height, fp, lp)
    cb = (cb0, cb1)
    ib = (ib0, ib1)
    gb = (gb0, gb1)
    pf = (pf0, pf1)
    pq = (pq0, pq1)
    sin = (sin0, sin1)
    sg = (sg0, sg1)
    sout = (sout0, sout1)

    sid = lax.axis_index("s")
    wid = sid * NC + lax.axis_index("c")

    @pl.when(sid == 0)
    def _stage():
        pltpu.sync_copy(tcat, shared)

    plsc.subcore_barrier()

    pltpu.sync_copy(fpe, fpev)
    pltpu.sync_copy(lpe, lpev)
    fpe_v = [fpev[pl.ds(16 * r, 16)] for r in range(SIZE // 16)]
    lpe_v = [lpev[pl.ds(16 * r, 16)] for r in range(SIZE // 16)]

    def fire_in(t, p):
        base = wid * PER_W + t * C
        for k in range(8):
            pltpu.async_copy(coords[k].at[pl.ds(base, C)], cb[p].at[k],
                             sin[p])

    def wait_in(p):
        for k in range(8):
            pltpu.make_async_copy(fp.at[pl.ds(0, C)], cb[p].at[k],
                                  sin[p]).wait()

    def do_idx(p):
        for j in range(C // LANES):
            s = pl.ds(j * LANES, LANES)
            pf[p][s] = cb[p][6, s]
            pq[p][s] = cb[p][7, s]
            for k in range(6):
                v = cb[p][k, s]
                f = jnp.minimum(v * _SCALES[k], _MAXF)
                f = jnp.maximum(f, 0.0)
                ib[p][pl.ds(k * C + j * LANES, LANES)] = (
                    f.astype(jnp.int32) + _OFFS[k])

    def fire_gather(p):
        pltpu.async_copy(shared.at[ib[p]], gb[p], sg[p])

    def wait_gather(p):
        pltpu.make_async_copy(tcat.at[pl.ds(0, 6 * C)], gb[p], sg[p]).wait()

    def do_page(p):
        def box_body(c, inner):
            idx16 = jnp.full((LANES,), c, jnp.int32)
            fpv = plsc.load_gather(pf[p], [idx16])
            lpv = plsc.load_gather(pq[p], [idx16])
            for k in range(6):
                for hh in range(2):
                    r = k * 2 + hh
                    g = gb[p][k * C + c, pl.ds(hh * 16, 16)]
                    gb[p][k * C + c, pl.ds(hh * 16, 16)] = (
                        g + fpv * fpe_v[r] + lpv * lpe_v[r])
            return inner
        lax.fori_loop(0, C, box_body, 0, unroll=False)

    def fire_out(t, p):
        base = wid * PER_W + t * C
        for k in range(6):
            pltpu.async_copy(gb[p].at[pl.ds(k * C, C)],
                             out.at[pl.ds(base, C), pl.ds(k * SUB, SUB)],
                             sout[p])

    def wait_out(p):
        for k in range(6):
            pltpu.make_async_copy(
                gb[p].at[pl.ds(k * C, C)],
                out.at[pl.ds(0, C), pl.ds(k * SUB, SUB)], sout[p]).wait()

    fire_in(0, 0)
    wait_in(0)
    fire_in(1, 1)
    do_idx(0)
    fire_gather(0)
    wait_in(1)
    fire_in(2, 0)
    do_idx(1)
    fire_gather(1)
    wait_gather(0)
    do_page(0)
    fire_out(0, 0)

    def steady(tt, carry):
        t0 = 2 * tt
        t1 = t0 + 1
        wait_in(0)
        fire_in(t0 + 1, 1)
        do_idx(0)
        wait_out(0)
        fire_gather(0)
        wait_gather(1)
        do_page(1)
        fire_out(t0 - 1, 1)
        wait_in(1)

        @pl.when(t1 + 1 < CHUNKS)
        def _():
            fire_in(t1 + 1, 0)
        do_idx(1)
        wait_out(1)
        fire_gather(1)
        wait_gather(0)
        do_page(0)
        fire_out(t0, 0)
        return carry

    lax.fori_loop(1, CHUNKS // 2, steady, 0, unroll=False)

    wait_gather(1)
    do_page(1)
    fire_out(CHUNKS - 1, 1)
    wait_out(0)
    wait_out(1)


@functools.partial(jax.jit, static_argnames=("interp",))
def _run(xmin, ymin, xmax, ymax, width, height, fp, lp,
         tcat, fpe, lpe, interp=False):
    mesh = plsc.VectorSubcoreMesh(core_axis_name="c", subcore_axis_name="s",
                                  num_cores=NC, num_subcores=NS)
    f = pl.kernel(
        _body,
        out_type=jax.ShapeDtypeStruct((N, SIZE), jnp.float32),
        mesh=mesh,
        scratch_types=(
            [pltpu.VMEM((8, C), jnp.float32) for _ in range(2)]
            + [pltpu.VMEM((C,), jnp.float32) for _ in range(4)]
            + [pltpu.VMEM((6 * C,), jnp.int32) for _ in range(2)]
            + [pltpu.VMEM((6 * C, SUB), jnp.float32) for _ in range(2)]
            + [pltpu.VMEM((SIZE,), jnp.float32) for _ in range(2)]
            + [pltpu.VMEM_SHARED((4 * N_POS, SUB), jnp.float32)]
            + [pltpu.SemaphoreType.DMA] * 6
        ),
        compiler_params=pltpu.CompilerParams(use_tc_tiling_on_sc=False,
                                             needs_layout_passes=False),
        interpret=interp,
    )
    return f(xmin, ymin, xmax, ymax, width, height, fp, lp, tcat, fpe, lpe)


def kernel(xmin, ymin, xmax, ymax, width, height, first_page, last_page,
           x_table, y_table, w_table, h_table, first_page_emb, last_page_emb):
    flat = [a.reshape(N) for a in (xmin, ymin, xmax, ymax, width, height,
                                   first_page, last_page)]
    tcat = jnp.concatenate([x_table, y_table, w_table, h_table], axis=0)
    out = _run(*flat, tcat, first_page_emb, last_page_emb)
    return out.reshape(B, L, SIZE)
